# baseline (speedup 1.0000x reference)
import jax
import jax.numpy as jnp
from jax import lax
from jax.experimental import pallas as pl

_BATCH = 16384
_NGROUPS = 32
_GSIZE = 4
_USED = _NGROUPS * _GSIZE

_NSTREAM = 16
_TC_BLK = 512
_OBLK = _NSTREAM * _TC_BLK
_GRID = _BATCH // _OBLK


def _tc_body(*refs):
    x_refs, o_ref = refs[:_NSTREAM], refs[_NSTREAM]
    k = lax.broadcasted_iota(jnp.int32, (_USED, _NGROUPS), 0)
    i = lax.broadcasted_iota(jnp.int32, (_USED, _NGROUPS), 1)
    w = jnp.where(k // _GSIZE == i, jnp.float32(1.0 / _GSIZE), jnp.float32(0.0))
    for q, x_ref in enumerate(x_refs):
        o_ref[q * _TC_BLK:(q + 1) * _TC_BLK, :] = jnp.dot(
            x_ref[...], w, preferred_element_type=jnp.float32,
            precision=lax.Precision.HIGHEST)


@jax.jit
def _pooled_mean(x):
    def in_map(q):
        return lambda i: (i * _NSTREAM + q, 0)

    return pl.pallas_call(
        _tc_body,
        grid=(_GRID,),
        in_specs=[pl.BlockSpec((_TC_BLK, _USED), in_map(q))
                  for q in range(_NSTREAM)],
        out_specs=pl.BlockSpec((_OBLK, _NGROUPS), lambda i: (i, 0)),
        out_shape=jax.ShapeDtypeStruct((_BATCH, _NGROUPS), jnp.float32),
    )(*([x] * _NSTREAM))


def kernel(gene_set_features):
    return _pooled_mean(gene_set_features)
